# baseline (speedup 1.0000x reference)
import functools

import jax
import jax.numpy as jnp
from jax import lax
from jax.experimental import pallas as pl
from jax.experimental.pallas import tpu as pltpu
from jax.experimental.pallas import tpu_sc as plsc

TOP_K = 8
ALPHA = 0.1
LANES = 16


def _gate_kernel(hs_ref, w_ref, idx_ref, wgt_ref, ssum_ref,
                 *, blocks_per_batch, seq, e):
    i = pl.program_id(0)
    b = i // blocks_per_batch

    @pl.when(i == 0)
    def _init():
        ssum_ref[...] = jnp.zeros_like(ssum_ref)

    hs = hs_ref[...]
    w = w_ref[...]
    logits = jax.lax.dot_general(
        hs, w, (((1,), (1,)), ((), ())), preferred_element_type=jnp.float32)
    m = jnp.max(logits, axis=1, keepdims=True)
    ex = jnp.exp(logits - m)
    scores = ex / jnp.sum(ex, axis=1, keepdims=True)

    iota = jax.lax.broadcasted_iota(jnp.int32, scores.shape, 1)
    bits = jax.lax.bitcast_convert_type(scores, jnp.int32)
    keys = (bits & jnp.int32(~(e - 1))) | (e - 1 - iota)
    minkey = jnp.int32(-2**31)
    vals = []
    idxs = []
    for _ in range(TOP_K):
        kmax = jnp.max(keys, axis=1, keepdims=True)
        keys = jnp.where(keys == kmax, minkey, keys)
        idxs.append(e - 1 - (kmax & (e - 1)))
        vals.append(jax.lax.bitcast_convert_type(
            kmax & jnp.int32(~(e - 1)), jnp.float32))

    topw = jnp.concatenate(vals, axis=1)
    topi = jnp.concatenate(idxs, axis=1)
    denom = jnp.sum(topw, axis=1, keepdims=True) + 1e-20
    wgt_ref[...] = topw / denom
    idx_ref[...] = topi

    ssum_part = jnp.sum(scores, axis=0, keepdims=True)
    ssum_ref[pl.ds(b, 1), :] += ssum_part


def _make_sc_hist(e, n_flat):
    info = plsc.get_sparse_core_info()
    nw = info.num_cores * info.num_subcores
    chunk = n_flat // nw
    vecs = chunk // LANES
    ec = e // LANES
    mesh = plsc.VectorSubcoreMesh(core_axis_name="c", subcore_axis_name="s")

    @functools.partial(
        pl.kernel, mesh=mesh,
        compiler_params=pltpu.CompilerParams(needs_layout_passes=False),
        out_type=jax.ShapeDtypeStruct((nw, e), jnp.float32),
        scratch_types=[
            pltpu.VMEM((chunk,), jnp.int32),
            pltpu.VMEM((LANES * e,), jnp.float32),
            pltpu.VMEM((e,), jnp.float32),
        ],
    )
    def sc_hist(idx_hbm, hists_out, chunk_v, hist_v, bins_v):
        c = lax.axis_index("c")
        s = lax.axis_index("s")
        w = s * info.num_cores + c
        zeros = jnp.zeros((LANES,), jnp.float32)
        for i in range(LANES * ec):
            hist_v[pl.ds(i * LANES, LANES)] = zeros
        pltpu.sync_copy(idx_hbm.at[pl.ds(w * chunk, chunk)], chunk_v)
        lane_off = lax.iota(jnp.int32, LANES) * e
        ones = jnp.ones((LANES,), jnp.float32)
        for i in range(vecs):
            vidx = chunk_v[pl.ds(i * LANES, LANES)]
            plsc.addupdate_scatter(hist_v, [vidx + lane_off], ones)
        for c4 in range(ec):
            acc = jnp.zeros((LANES,), jnp.float32)
            for l in range(LANES):
                acc = acc + hist_v[pl.ds(l * e + c4 * LANES, LANES)]
            bins_v[pl.ds(c4 * LANES, LANES)] = acc
        pltpu.sync_copy(bins_v, hists_out.at[w])

    return sc_hist, nw, chunk


def _stats_kernel(hists_ref, ssum_ref, loads_ref, aux_ref, vio_ref, imb_ref,
                  util_ref, ratio_ref, *, bsz, seq, e, tiles_per_batch):
    hists = hists_ref[...]
    ssum = ssum_ref[...]
    loads = jnp.sum(hists, axis=0, keepdims=True)
    loads_ref[...] = loads
    total = jnp.sum(loads)
    expected = total / e
    mean = total / e
    maxl = jnp.max(loads)
    vio_ref[...] = ((maxl - expected) / expected).reshape(1, 1)
    var = jnp.sum((loads - mean) ** 2) / (e - 1)
    imb_ref[...] = (jnp.sqrt(var) / mean).reshape(1, 1)
    util_ref[...] = (jnp.sum((loads > 0).astype(jnp.float32)) / e).reshape(1, 1)
    minl = jnp.min(jnp.where(loads > 0, loads, jnp.inf))
    ratio_ref[...] = (maxl / minl).reshape(1, 1)
    nw = hists.shape[0]
    counts = jnp.sum(hists.reshape(bsz, tiles_per_batch, e), axis=1)
    ce = counts / (seq * TOP_K / e)
    smean = ssum / seq
    aux_ref[...] = (jnp.sum(ce * smean) / bsz * ALPHA).reshape(1, 1)


def kernel(hidden_states, weight):
    bsz, seq, h = hidden_states.shape
    e = weight.shape[0]
    blk = 1024 if seq % 1024 == 0 else seq
    blocks_per_batch = seq // blk
    num_blocks = bsz * blocks_per_batch
    hs = hidden_states.reshape(bsz * seq, h)

    topi, topw, ssum = pl.pallas_call(
        functools.partial(_gate_kernel, blocks_per_batch=blocks_per_batch,
                          seq=seq, e=e),
        grid=(num_blocks,),
        in_specs=[
            pl.BlockSpec((blk, h), lambda i: (i, 0)),
            pl.BlockSpec((e, h), lambda i: (0, 0)),
        ],
        out_specs=(
            pl.BlockSpec((blk, TOP_K), lambda i: (i, 0)),
            pl.BlockSpec((blk, TOP_K), lambda i: (i, 0)),
            pl.BlockSpec((bsz, e), lambda i: (0, 0)),
        ),
        out_shape=(
            jax.ShapeDtypeStruct((bsz * seq, TOP_K), jnp.int32),
            jax.ShapeDtypeStruct((bsz * seq, TOP_K), jnp.float32),
            jax.ShapeDtypeStruct((bsz, e), jnp.float32),
        ),
    )(hs, weight)

    sc_hist, nw, chunk = _make_sc_hist(e, bsz * seq * TOP_K)
    hists = sc_hist(topi.reshape(-1))
    tiles_per_batch = nw // bsz

    loads, aux, vio, imb, util, ratio = pl.pallas_call(
        functools.partial(_stats_kernel, bsz=bsz, seq=seq, e=e,
                          tiles_per_batch=tiles_per_batch),
        out_shape=(
            jax.ShapeDtypeStruct((1, e), jnp.float32),
            jax.ShapeDtypeStruct((1, 1), jnp.float32),
            jax.ShapeDtypeStruct((1, 1), jnp.float32),
            jax.ShapeDtypeStruct((1, 1), jnp.float32),
            jax.ShapeDtypeStruct((1, 1), jnp.float32),
            jax.ShapeDtypeStruct((1, 1), jnp.float32),
        ),
    )(hists, ssum)

    return (
        topi,
        topw,
        aux[0, 0],
        loads[0],
        vio[0, 0],
        imb[0, 0],
        util[0, 0],
        ratio[0, 0],
    )
